# Initial kernel scaffold; baseline (speedup 1.0000x reference)
import jax
import jax.numpy as jnp
from jax.experimental import pallas as pl


def kernel(seq, edge_index, W_fc, W_gat, a_src, a_dst, b_conv, bias, prelu_a):
    raise NotImplementedError("write your pallas kernel here")


# speedup vs baseline: 22.1365x; 22.1365x over previous
import functools

import jax
import jax.numpy as jnp
from jax import lax
from jax.experimental import pallas as pl
from jax.experimental.pallas import tpu as pltpu
from jax.experimental.pallas import tpu_sc as plsc

N = 10000
NP = 10240
E = 320000
D = 128
DH = 144
NC = 2
NS = 16
NW = NC * NS
EPT = E // NW
C = 80
CHUNKS = EPT // C
RPT = NP // NS
ZROWS = RPT // 10


def _tc_pre(seq_ref, wfc_ref, wgat_ref, a2_ref, h_ref, asd_ref):
    wc = jnp.dot(wfc_ref[...], wgat_ref[...], preferred_element_type=jnp.float32)
    hb = jnp.dot(seq_ref[...], wc, preferred_element_type=jnp.float32)
    col = jax.lax.broadcasted_iota(jnp.int32, (seq_ref.shape[0], DH - D), 1)
    pad = jnp.where(col == 0, 1.0, 0.0).astype(jnp.float32)
    h_ref[...] = jnp.concatenate([hb, pad], axis=1)
    asd_ref[...] = jnp.dot(hb, a2_ref[...], preferred_element_type=jnp.float32)


def _tc_post(acc_ref, asd_ref, h_ref, bc_ref, b_ref, pa_ref, o_ref):
    es = asd_ref[:, 0:1] + asd_ref[:, 1:2]
    es = jnp.where(es > 0, es, 0.2 * es)
    ws = jnp.exp(es)
    acc = acc_ref[0] + acc_ref[1]
    num = acc[:, 0:D] + ws * h_ref[:, 0:D]
    den = acc[:, D:D + 1] + ws + 1e-16
    out = num / den + (bc_ref[...] + b_ref[...])
    pa = pa_ref[0, 0]
    o_ref[...] = jnp.where(out >= 0, out, pa * out)


def _sc_edge_kernel():
    mesh = plsc.VectorSubcoreMesh(core_axis_name="c", subcore_axis_name="s")

    @functools.partial(
        pl.kernel,
        out_type=jax.ShapeDtypeStruct((NC, NP, DH), jnp.float32),
        mesh=mesh,
        compiler_params=pltpu.CompilerParams(
            use_tc_tiling_on_sc=False, needs_layout_passes=False),
        scratch_types=[
            pltpu.VMEM((C,), jnp.int32),
            pltpu.VMEM((C,), jnp.int32),
            pltpu.VMEM((C,), jnp.float32),
            pltpu.VMEM((C,), jnp.float32),
            pltpu.VMEM((C,), jnp.float32),
            pltpu.VMEM((C, DH), jnp.float32),
            pltpu.VMEM((ZROWS, DH), jnp.float32),
            pltpu.VMEM_SHARED((NP, DH), jnp.float32),
            pltpu.SemaphoreType.DMA,
            pltpu.SemaphoreType.DMA,
            pltpu.SemaphoreType.DMA,
        ],
    )
    def k(src_hbm, dst_hbm, as_hbm, ad_hbm, h_hbm, acc_out,
          src_v, dst_v, asg_v, adg_v, w_v, rows_v, zb_v, acc_sh,
          sem1, sem2, sem3):
        cid = lax.axis_index("c")
        sid = lax.axis_index("s")
        wid = cid * NS + sid

        nzchunks = ZROWS * (DH // 16)

        def zb_body(i, carry):
            zb_v[i // (DH // 16), pl.ds((i % (DH // 16)) * 16, 16)] = (
                jnp.zeros((16,), jnp.float32))
            return carry

        lax.fori_loop(0, nzchunks, zb_body, 0)
        for r in range(10):
            pltpu.sync_copy(zb_v, acc_sh.at[pl.ds(sid * RPT + r * ZROWS, ZROWS)])
        plsc.subcore_barrier()

        ebase = wid * EPT

        def chunk_body(kk, carry):
            base = ebase + kk * C
            pltpu.sync_copy(src_hbm.at[pl.ds(base, C)], src_v)
            pltpu.sync_copy(dst_hbm.at[pl.ds(base, C)], dst_v)
            cp3 = pltpu.async_copy(h_hbm.at[src_v], rows_v, sem3)
            cp1 = pltpu.async_copy(as_hbm.at[src_v], asg_v, sem1)
            cp2 = pltpu.async_copy(ad_hbm.at[dst_v], adg_v, sem2)
            cp1.wait()
            cp2.wait()
            for v in range(C // 16):
                av = asg_v[pl.ds(v * 16, 16)]
                bv = adg_v[pl.ds(v * 16, 16)]
                e = av + bv
                e = jnp.where(e > 0, e, 0.2 * e)
                w_v[pl.ds(v * 16, 16)] = jnp.exp(e)
            cp3.wait()

            def scale_body(i, carry2):
                wb = plsc.load_gather(w_v, [jnp.full((16,), i, jnp.int32)])
                for j in range(DH // 16):
                    rows_v[i, pl.ds(j * 16, 16)] = rows_v[i, pl.ds(j * 16, 16)] * wb
                return carry2

            lax.fori_loop(0, C, scale_body, 0)
            pltpu.sync_copy(rows_v, acc_sh.at[dst_v], add=True)
            return carry

        lax.fori_loop(0, CHUNKS, chunk_body, 0)
        plsc.subcore_barrier()
        pltpu.sync_copy(acc_sh.at[pl.ds(sid * RPT, RPT)],
                        acc_out.at[cid, pl.ds(sid * RPT, RPT)])

    return k


_sc_edge = _sc_edge_kernel()


@jax.jit
def kernel(seq, edge_index, W_fc, W_gat, a_src, a_dst, b_conv, bias, prelu_a):
    a2 = jnp.stack([a_src, a_dst], axis=1)
    src = edge_index[0].astype(jnp.int32)
    dst = edge_index[1].astype(jnp.int32)
    seq_p = jnp.pad(seq, ((0, NP - N), (0, 0)))

    h144, asd = pl.pallas_call(
        _tc_pre,
        out_shape=[
            jax.ShapeDtypeStruct((NP, DH), jnp.float32),
            jax.ShapeDtypeStruct((NP, 2), jnp.float32),
        ],
    )(seq_p, W_fc, W_gat, a2)

    as_arr = asd[:, 0]
    ad_arr = asd[:, 1]
    acc = _sc_edge(src, dst, as_arr, ad_arr, h144)

    B = 640
    out = pl.pallas_call(
        _tc_post,
        grid=(NP // B,),
        in_specs=[
            pl.BlockSpec((NC, B, DH), lambda i: (0, i, 0)),
            pl.BlockSpec((B, 2), lambda i: (i, 0)),
            pl.BlockSpec((B, DH), lambda i: (i, 0)),
            pl.BlockSpec((1, D), lambda i: (0, 0)),
            pl.BlockSpec((1, D), lambda i: (0, 0)),
            pl.BlockSpec((1, 1), lambda i: (0, 0)),
        ],
        out_specs=pl.BlockSpec((B, D), lambda i: (i, 0)),
        out_shape=jax.ShapeDtypeStruct((NP, D), jnp.float32),
    )(acc, asd, h144, b_conv.reshape(1, D), bias.reshape(1, D),
      prelu_a.reshape(1, 1))
    return out[:N]
